# baseline (speedup 1.0000x reference)
import functools
import math

import jax
import jax.numpy as jnp
from jax import lax
from jax.experimental import pallas as pl
from jax.experimental.pallas import tpu as pltpu
from jax.experimental.pallas import tpu_sc as plsc

_N = 4096
_E = 4096
_NUM_SCALARS = 256
_NUM_RADIAL = 32
_CUTOFF = 10.0
_HEAD_DIM = _NUM_SCALARS
_INV_SQRT_HD = 1.0 / math.sqrt(float(_HEAD_DIM))
_KATT = _INV_SQRT_HD / math.sqrt(3.0)

_NCORES = 2
_NSUB = 16
_NW = _NCORES * _NSUB
_EPW = _E // _NW
_NSLICE = _N // _NSUB

_HIGH = lax.Precision.HIGHEST


def _prep_body(vec9_ref, evt_ref, mqk_ref, wep_ref, bep_ref,
               freqs_ref, t_ref, eatt_ref):
    ev = evt_ref[...]
    d = jnp.sqrt(jnp.sum(ev * ev, axis=0, keepdims=True))
    num = jnp.sin(freqs_ref[...] * d)
    den = jnp.maximum(d, 1e-08)
    cut = 0.5 * (jnp.cos(d * (math.pi / _CUTOFF)) + 1.0)
    cut = cut * (d < _CUTOFF).astype(jnp.float32)
    bes = (num / den) * cut
    eatt = lax.dot_general(wep_ref[...], bes,
                           (((1,), (0,)), ((), ())), precision=_HIGH)
    eatt_ref[...] = (eatt + bep_ref[0, 0]) * _INV_SQRT_HD
    t_ref[...] = lax.dot_general(vec9_ref[...], mqk_ref[...],
                                 (((1,), (0,)), ((), ())), precision=_HIGH)


_prep_call = pl.pallas_call(
    _prep_body,
    out_shape=[
        jax.ShapeDtypeStruct((_N, 128), jnp.float32),
        jax.ShapeDtypeStruct((1, _E), jnp.float32),
    ],
)


def _sc_body(t_hbm, ei_hbm, eatt_hbm, u2_hbm,
             idx_r, idx_c, eatt_v, ew_v, rows_a, rows_b, zbuf, u_sh,
             sem_a, sem_b):
    c = lax.axis_index("c")
    s = lax.axis_index("s")
    wid = s * _NCORES + c
    base = wid * _EPW

    z16 = jnp.zeros((16,), jnp.float32)
    for i in range(_NSLICE // 16):
        zbuf[pl.ds(i * 16, 16)] = z16
    pltpu.sync_copy(zbuf, u_sh.at[pl.ds(s * _NSLICE, _NSLICE)])

    pltpu.sync_copy(ei_hbm.at[0, pl.ds(base, _EPW)], idx_r)
    pltpu.sync_copy(ei_hbm.at[1, pl.ds(base, _EPW)], idx_c)
    pltpu.sync_copy(eatt_hbm.at[0, pl.ds(base, _EPW)], eatt_v)
    cp_a = pltpu.async_copy(t_hbm.at[idx_r], rows_a, sem_a)
    cp_b = pltpu.async_copy(t_hbm.at[idx_c], rows_b, sem_b)
    cp_a.wait()
    cp_b.wait()

    plsc.subcore_barrier()

    for k in range(_EPW // 16):
        lidx = lax.iota(jnp.int32, 16) + (k * 16)
        acc = jnp.zeros((16,), jnp.float32)
        for j in range(9):
            a = plsc.load_gather(rows_a, [lidx, jnp.full((16,), j, jnp.int32)])
            b = plsc.load_gather(rows_b, [lidx, jnp.full((16,), j + 16, jnp.int32)])
            acc = acc + a * b
        z = acc * _KATT + eatt_v[pl.ds(k * 16, 16)]
        ew_v[pl.ds(k * 16, 16)] = jnp.exp(z)

    pltpu.sync_copy(ew_v, u_sh.at[idx_c], add=True)

    plsc.subcore_barrier()
    pltpu.sync_copy(u_sh.at[pl.ds(s * _NSLICE, _NSLICE)],
                    u2_hbm.at[c, pl.ds(s * _NSLICE, _NSLICE)])


_sc_call = functools.partial(
    pl.kernel,
    out_type=jax.ShapeDtypeStruct((_NCORES, _N), jnp.float32),
    mesh=plsc.VectorSubcoreMesh(core_axis_name="c", subcore_axis_name="s"),
    compiler_params=pltpu.CompilerParams(needs_layout_passes=False),
    scratch_types=[
        pltpu.VMEM((_EPW,), jnp.int32),
        pltpu.VMEM((_EPW,), jnp.int32),
        pltpu.VMEM((_EPW,), jnp.float32),
        pltpu.VMEM((_EPW,), jnp.float32),
        pltpu.VMEM((_EPW, 128), jnp.float32),
        pltpu.VMEM((_EPW, 128), jnp.float32),
        pltpu.VMEM((_NSLICE,), jnp.float32),
        pltpu.VMEM_SHARED((_N,), jnp.float32),
        pltpu.SemaphoreType.DMA,
        pltpu.SemaphoreType.DMA,
    ],
)(_sc_body)


_FCH = 8
_FCB = _N // _FCH


def _finish_body(u2_ref, scal_hbm, vec9_ref, wvs_ref, bvs_ref, wos_ref,
                 bos_ref, m2_ref, bov9_ref, gam_ref, bet_ref,
                 sout_hbm, vrow_ref, scal_v, outb0, outb1,
                 sem_in, sem_o0, sem_o1):
    for c in range(_FCH):
        sl = pl.ds(c * _FCB, _FCB)
        pltpu.make_async_copy(scal_hbm.at[sl, :], scal_v.at[sl, :],
                              sem_in).start()

    acc = jnp.zeros((1, _NUM_SCALARS), jnp.float32)
    for c in range(_FCH):
        sl = pl.ds(c * _FCB, _FCB)
        pltpu.make_async_copy(scal_hbm.at[sl, :], scal_v.at[sl, :],
                              sem_in).wait()
        u_blk = u2_ref[0:1, sl] + u2_ref[1:2, sl]
        acc = acc + lax.dot_general(u_blk, scal_v[sl, :],
                                    (((1,), (0,)), ((), ())),
                                    precision=_HIGH)

    u = u2_ref[0:1, :] + u2_ref[1:2, :]
    s_tot = jnp.sum(u)
    inv = 1.0 / s_tot
    r_v = lax.dot_general(u * inv, vec9_ref[...],
                          (((1,), (0,)), ((), ())), precision=_HIGH)
    sagg = lax.dot_general(acc * inv, wvs_ref[...],
                           (((1,), (1,)), ((), ())), precision=_HIGH) + bvs_ref[...]
    srow = lax.dot_general(sagg, wos_ref[...],
                           (((1,), (1,)), ((), ())), precision=_HIGH) + bos_ref[...]
    vrow_ref[...] = lax.dot_general(r_v, m2_ref[...],
                                    (((1,), (0,)), ((), ())),
                                    precision=_HIGH) + bov9_ref[...]

    bufs = (outb0, outb1)
    sems = (sem_o0, sem_o1)
    for c in range(_FCH):
        sl = pl.ds(c * _FCB, _FCB)
        buf = bufs[c % 2]
        if c >= 2:
            old = pl.ds((c - 2) * _FCB, _FCB)
            pltpu.make_async_copy(buf, sout_hbm.at[old, :], sems[c % 2]).wait()
        x = scal_v[sl, :] + srow
        mu = jnp.mean(x, axis=1, keepdims=True)
        xc = x - mu
        var = jnp.mean(xc * xc, axis=1, keepdims=True)
        buf[...] = (xc * jax.lax.rsqrt(var + 1e-05) * gam_ref[...]
                    + bet_ref[...])
        pltpu.make_async_copy(buf, sout_hbm.at[sl, :], sems[c % 2]).start()
    for c in range(_FCH - 2, _FCH):
        sl = pl.ds(c * _FCB, _FCB)
        pltpu.make_async_copy(bufs[c % 2], sout_hbm.at[sl, :],
                              sems[c % 2]).wait()


_finish_call = pl.pallas_call(
    _finish_body,
    in_specs=[
        pl.BlockSpec((2, _N), lambda: (0, 0)),
        pl.BlockSpec(memory_space=pl.ANY),
        pl.BlockSpec((_N, 9), lambda: (0, 0)),
        pl.BlockSpec((_NUM_SCALARS, _NUM_SCALARS), lambda: (0, 0)),
        pl.BlockSpec((1, _NUM_SCALARS), lambda: (0, 0)),
        pl.BlockSpec((_NUM_SCALARS, _NUM_SCALARS), lambda: (0, 0)),
        pl.BlockSpec((1, _NUM_SCALARS), lambda: (0, 0)),
        pl.BlockSpec((9, 9), lambda: (0, 0)),
        pl.BlockSpec((1, 9), lambda: (0, 0)),
        pl.BlockSpec((1, _NUM_SCALARS), lambda: (0, 0)),
        pl.BlockSpec((1, _NUM_SCALARS), lambda: (0, 0)),
    ],
    out_specs=[
        pl.BlockSpec(memory_space=pl.ANY),
        pl.BlockSpec((1, 9), lambda: (0, 0)),
    ],
    out_shape=[
        jax.ShapeDtypeStruct((_N, _NUM_SCALARS), jnp.float32),
        jax.ShapeDtypeStruct((1, 9), jnp.float32),
    ],
    scratch_shapes=[
        pltpu.VMEM((_N, _NUM_SCALARS), jnp.float32),
        pltpu.VMEM((_FCB, _NUM_SCALARS), jnp.float32),
        pltpu.VMEM((_FCB, _NUM_SCALARS), jnp.float32),
        pltpu.SemaphoreType.DMA,
        pltpu.SemaphoreType.DMA,
        pltpu.SemaphoreType.DMA,
    ],
)


def kernel(scalars, vectors, edge_index, edge_vec, W_qs, b_qs, W_ks, b_ks,
           W_vs, b_vs, W_qv, W_kv, W_vv, W_os, b_os, W_ov, b_ov, W_ep, b_ep,
           gamma_s, beta_s):
    del W_qs, b_qs, W_ks, b_ks

    vec9 = vectors.reshape(_N, 9)

    eye3 = jnp.eye(3, dtype=jnp.float32)
    pad7 = jnp.zeros((9, 7), dtype=jnp.float32)
    pad103 = jnp.zeros((9, 103), dtype=jnp.float32)
    mqk = jnp.concatenate([jnp.kron(eye3, W_qv.T), pad7,
                           jnp.kron(eye3, W_kv.T), pad103], axis=1)
    m2 = jnp.kron(W_ov.T, W_vv.T)
    bov9 = jnp.repeat(b_ov, 3).reshape(1, 9)
    freqs = (jnp.arange(1, _NUM_RADIAL + 1, dtype=jnp.float32)
             * (math.pi / _CUTOFF)).reshape(_NUM_RADIAL, 1)

    t_tab, eatt = _prep_call(vec9, edge_vec.T, mqk,
                             W_ep.reshape(1, _NUM_RADIAL),
                             b_ep.reshape(1, 1), freqs)

    u2 = _sc_call(t_tab, edge_index, eatt)

    sout, vrow = _finish_call(u2, scalars, vec9,
                              W_vs, b_vs.reshape(1, _NUM_SCALARS),
                              W_os, b_os.reshape(1, _NUM_SCALARS),
                              m2, bov9,
                              gamma_s.reshape(1, _NUM_SCALARS),
                              beta_s.reshape(1, _NUM_SCALARS))
    return (sout, vectors + vrow.reshape(1, 3, 3))


# speedup vs baseline: 1.0556x; 1.0556x over previous
import functools
import math

import jax
import jax.numpy as jnp
from jax import lax
from jax.experimental import pallas as pl
from jax.experimental.pallas import tpu as pltpu
from jax.experimental.pallas import tpu_sc as plsc

_N = 4096
_E = 4096
_NUM_SCALARS = 256
_NUM_RADIAL = 32
_CUTOFF = 10.0
_HEAD_DIM = _NUM_SCALARS
_INV_SQRT_HD = 1.0 / math.sqrt(float(_HEAD_DIM))
_KATT = _INV_SQRT_HD / math.sqrt(3.0)

_NCORES = 2
_NSUB = 16
_NW = _NCORES * _NSUB
_EPW = _E // _NW
_NSLICE = _N // _NSUB

_HIGH = lax.Precision.HIGHEST


def _prep_body(vec9_ref, evt_ref, mqk_ref, wep_ref, bep_ref,
               freqs_ref, t_ref, eatt_ref):
    ev = evt_ref[...]
    d = jnp.sqrt(jnp.sum(ev * ev, axis=0, keepdims=True))
    num = jnp.sin(freqs_ref[...] * d)
    den = jnp.maximum(d, 1e-08)
    cut = 0.5 * (jnp.cos(d * (math.pi / _CUTOFF)) + 1.0)
    cut = cut * (d < _CUTOFF).astype(jnp.float32)
    bes = (num / den) * cut
    eatt = lax.dot_general(wep_ref[...], bes,
                           (((1,), (0,)), ((), ())), precision=_HIGH)
    eatt_ref[...] = (eatt + bep_ref[0, 0]) * _INV_SQRT_HD
    t_ref[...] = lax.dot_general(vec9_ref[...], mqk_ref[...],
                                 (((1,), (0,)), ((), ())), precision=_HIGH)


_prep_call = pl.pallas_call(
    _prep_body,
    out_shape=[
        jax.ShapeDtypeStruct((_N, 128), jnp.float32),
        jax.ShapeDtypeStruct((1, _E), jnp.float32),
    ],
)


def _sc_body(t_hbm, ei_hbm, eatt_hbm, u2_hbm,
             idx_r, idx_c, eatt_v, ew_v, rows_a, rows_b, zbuf, u_sh,
             sem_a, sem_b):
    c = lax.axis_index("c")
    s = lax.axis_index("s")
    wid = s * _NCORES + c
    base = wid * _EPW

    z16 = jnp.zeros((16,), jnp.float32)
    for i in range(_NSLICE // 16):
        zbuf[pl.ds(i * 16, 16)] = z16
    pltpu.sync_copy(zbuf, u_sh.at[pl.ds(s * _NSLICE, _NSLICE)])

    pltpu.sync_copy(ei_hbm.at[0, pl.ds(base, _EPW)], idx_r)
    pltpu.sync_copy(ei_hbm.at[1, pl.ds(base, _EPW)], idx_c)
    pltpu.sync_copy(eatt_hbm.at[0, pl.ds(base, _EPW)], eatt_v)
    cp_a = pltpu.async_copy(t_hbm.at[idx_r], rows_a, sem_a)
    cp_b = pltpu.async_copy(t_hbm.at[idx_c], rows_b, sem_b)
    cp_a.wait()
    cp_b.wait()

    plsc.subcore_barrier()

    for k in range(_EPW // 16):
        lidx = lax.iota(jnp.int32, 16) + (k * 16)
        acc = jnp.zeros((16,), jnp.float32)
        for j in range(9):
            a = plsc.load_gather(rows_a, [lidx, jnp.full((16,), j, jnp.int32)])
            b = plsc.load_gather(rows_b, [lidx, jnp.full((16,), j + 16, jnp.int32)])
            acc = acc + a * b
        z = acc * _KATT + eatt_v[pl.ds(k * 16, 16)]
        ew_v[pl.ds(k * 16, 16)] = jnp.exp(z)

    pltpu.sync_copy(ew_v, u_sh.at[idx_c], add=True)

    plsc.subcore_barrier()
    pltpu.sync_copy(u_sh.at[pl.ds(s * _NSLICE, _NSLICE)],
                    u2_hbm.at[c, pl.ds(s * _NSLICE, _NSLICE)])


_sc_call = functools.partial(
    pl.kernel,
    out_type=jax.ShapeDtypeStruct((_NCORES, _N), jnp.float32),
    mesh=plsc.VectorSubcoreMesh(core_axis_name="c", subcore_axis_name="s"),
    compiler_params=pltpu.CompilerParams(needs_layout_passes=False),
    scratch_types=[
        pltpu.VMEM((_EPW,), jnp.int32),
        pltpu.VMEM((_EPW,), jnp.int32),
        pltpu.VMEM((_EPW,), jnp.float32),
        pltpu.VMEM((_EPW,), jnp.float32),
        pltpu.VMEM((_EPW, 128), jnp.float32),
        pltpu.VMEM((_EPW, 128), jnp.float32),
        pltpu.VMEM((_NSLICE,), jnp.float32),
        pltpu.VMEM_SHARED((_N,), jnp.float32),
        pltpu.SemaphoreType.DMA,
        pltpu.SemaphoreType.DMA,
    ],
)(_sc_body)


def _finish_body(u2_ref, scal_ref, vec9_ref, wvs_ref, bvs_ref, wos_ref,
                 bos_ref, m2_ref, bov9_ref, gam_ref, bet_ref,
                 sout_ref, vrow_ref):
    u = u2_ref[0:1, :] + u2_ref[1:2, :]
    s_tot = jnp.sum(u)
    un = u * (1.0 / s_tot)
    r_s = lax.dot_general(un, scal_ref[...],
                          (((1,), (0,)), ((), ())), precision=_HIGH)
    r_v = lax.dot_general(un, vec9_ref[...],
                          (((1,), (0,)), ((), ())), precision=_HIGH)
    sagg = lax.dot_general(r_s, wvs_ref[...],
                           (((1,), (1,)), ((), ())), precision=_HIGH) + bvs_ref[...]
    srow = lax.dot_general(sagg, wos_ref[...],
                           (((1,), (1,)), ((), ())), precision=_HIGH) + bos_ref[...]
    x = scal_ref[...] + srow
    mu = jnp.mean(x, axis=1, keepdims=True)
    xc = x - mu
    var = jnp.mean(xc * xc, axis=1, keepdims=True)
    sout_ref[...] = xc * jax.lax.rsqrt(var + 1e-05) * gam_ref[...] + bet_ref[...]
    vrow_ref[...] = lax.dot_general(r_v, m2_ref[...],
                                    (((1,), (0,)), ((), ())),
                                    precision=_HIGH) + bov9_ref[...]


_finish_call = pl.pallas_call(
    _finish_body,
    out_shape=[
        jax.ShapeDtypeStruct((_N, _NUM_SCALARS), jnp.float32),
        jax.ShapeDtypeStruct((1, 9), jnp.float32),
    ],
)


def kernel(scalars, vectors, edge_index, edge_vec, W_qs, b_qs, W_ks, b_ks,
           W_vs, b_vs, W_qv, W_kv, W_vv, W_os, b_os, W_ov, b_ov, W_ep, b_ep,
           gamma_s, beta_s):
    del W_qs, b_qs, W_ks, b_ks

    vec9 = vectors.reshape(_N, 9)

    eye3 = jnp.eye(3, dtype=jnp.float32)
    pad7 = jnp.zeros((9, 7), dtype=jnp.float32)
    pad103 = jnp.zeros((9, 103), dtype=jnp.float32)
    mqk = jnp.concatenate([jnp.kron(eye3, W_qv.T), pad7,
                           jnp.kron(eye3, W_kv.T), pad103], axis=1)
    m2 = jnp.kron(W_ov.T, W_vv.T)
    bov9 = jnp.repeat(b_ov, 3).reshape(1, 9)
    freqs = (jnp.arange(1, _NUM_RADIAL + 1, dtype=jnp.float32)
             * (math.pi / _CUTOFF)).reshape(_NUM_RADIAL, 1)

    t_tab, eatt = _prep_call(vec9, edge_vec.T, mqk,
                             W_ep.reshape(1, _NUM_RADIAL),
                             b_ep.reshape(1, 1), freqs)

    u2 = _sc_call(t_tab, edge_index, eatt)

    sout, vrow = _finish_call(u2, scalars, vec9,
                              W_vs, b_vs.reshape(1, _NUM_SCALARS),
                              W_os, b_os.reshape(1, _NUM_SCALARS),
                              m2, bov9,
                              gamma_s.reshape(1, _NUM_SCALARS),
                              beta_s.reshape(1, _NUM_SCALARS))
    return (sout, vectors + vrow.reshape(1, 3, 3))
